# baseline (speedup 1.0000x reference)
import functools

import jax
import jax.numpy as jnp
from jax import lax
from jax.experimental import pallas as pl
from jax.experimental.pallas import tpu as pltpu
from jax.experimental.pallas import tpu_sc as plsc

BATCH = 16384
D = 32
NC = 2
NS = 16
NW = NC * NS
BPW = BATCH // NW
CHUNK = 128
NCH = BPW // CHUNK
L = 16
GPC = CHUNK // L
TBLK = 128
RPB = TBLK // D


def _rsqrt(x):
    i = plsc.bitcast(x, jnp.int32)
    i = jnp.int32(0x5F3759DF) - (i >> 1)
    y = plsc.bitcast(i, jnp.float32)
    for _ in range(3):
        y = y * (jnp.float32(1.5) - jnp.float32(0.5) * x * y * y)
    return y


def _body(user_hbm, item_hbm, utab_hbm, itab_hbm, out_hbm,
          uidx, iidx, uq, iq, ub0, ub1, ib0, ib1, outv, usem, isem):
    wid = lax.axis_index("s") * NC + lax.axis_index("c")

    for t in range(NCH):
        sl = pl.ds(t * CHUNK, CHUNK)
        pltpu.sync_copy(user_hbm.at[wid * NCH + t], uidx.at[sl])
        pltpu.sync_copy(item_hbm.at[wid * NCH + t], iidx.at[sl])

    def qstep(g, carry):
        sl = pl.ds(g * L, L)
        uq[sl] = uidx[sl] >> 2
        iq[sl] = iidx[sl] >> 2
        return carry
    lax.fori_loop(0, BPW // L, qstep, 0)

    ubufs = (ub0, ub1)
    ibufs = (ib0, ib1)

    def start(c):
        b = c & 1
        sl = pl.ds(c * CHUNK, CHUNK)
        return (pltpu.async_copy(utab_hbm.at[uq.at[sl]], ubufs[b], usem),
                pltpu.async_copy(itab_hbm.at[iq.at[sl]], ibufs[b], isem))

    inflight = start(0)
    for c in range(NCH):
        cu, ci = inflight
        cu.wait()
        ci.wait()
        if c + 1 < NCH:
            inflight = start(c + 1)
        ub = ubufs[c & 1]
        ib = ibufs[c & 1]

        def step(g, carry, c=c, ub=ub, ib=ib):
            k = lax.iota(jnp.int32, L) + g * L
            sl = pl.ds(c * CHUNK + g * L, L)
            ucol = (uidx[sl] & 3) << 5
            icol = (iidx[sl] & 3) << 5
            dot = jnp.zeros((L,), jnp.float32)
            n2u = jnp.zeros((L,), jnp.float32)
            n2v = jnp.zeros((L,), jnp.float32)
            for j in range(D):
                u = plsc.load_gather(ub, [k, ucol + j])
                v = plsc.load_gather(ib, [k, icol + j])
                dot = dot + u * v
                n2u = n2u + u * u
                n2v = n2v + v * v
            r = (dot
                 * _rsqrt(jnp.maximum(n2u, jnp.float32(1e-24)))
                 * _rsqrt(jnp.maximum(n2v, jnp.float32(1e-24))))
            outv[sl] = r
            return carry

        lax.fori_loop(0, GPC, step, 0)

    pltpu.sync_copy(outv, out_hbm.at[pl.ds(wid * BPW, BPW)])


_cosine = functools.partial(
    pl.kernel,
    out_type=jax.ShapeDtypeStruct((BATCH,), jnp.float32),
    mesh=plsc.VectorSubcoreMesh(core_axis_name="c", subcore_axis_name="s"),
    compiler_params=pltpu.CompilerParams(
        needs_layout_passes=False, use_tc_tiling_on_sc=False),
    scratch_types=[
        pltpu.VMEM((BPW,), jnp.int32),
        pltpu.VMEM((BPW,), jnp.int32),
        pltpu.VMEM((BPW,), jnp.int32),
        pltpu.VMEM((BPW,), jnp.int32),
        pltpu.VMEM((CHUNK, TBLK), jnp.float32),
        pltpu.VMEM((CHUNK, TBLK), jnp.float32),
        pltpu.VMEM((CHUNK, TBLK), jnp.float32),
        pltpu.VMEM((CHUNK, TBLK), jnp.float32),
        pltpu.VMEM((BPW,), jnp.float32),
        pltpu.SemaphoreType.DMA,
        pltpu.SemaphoreType.DMA,
    ],
)(_body)


def kernel(user, item, user_table, item_table):
    ut = user_table.reshape(-1, TBLK)
    it = item_table.reshape(-1, TBLK)
    u2 = user.astype(jnp.int32).reshape(NW * NCH, CHUNK)
    i2 = item.astype(jnp.int32).reshape(NW * NCH, CHUNK)
    return _cosine(u2, i2, ut, it)
